# baseline (speedup 1.0000x reference)
import jax
import jax.numpy as jnp
from jax.experimental import pallas as pl

_ROWS = 256
_COLS = 128


def _split(a):
    hi = a.astype(jnp.bfloat16).astype(jnp.float32)
    return hi, a - hi


def _cumsum_body(x_ref, o_ref):
    x = x_ref[:]

    ii = jax.lax.broadcasted_iota(jnp.int32, (_COLS, 2 * _COLS), 0)
    jj = jax.lax.broadcasted_iota(jnp.int32, (_COLS, 2 * _COLS), 1)
    uj = ((ii <= jj) | (jj >= _COLS)).astype(jnp.float32)
    ujuj = jnp.concatenate([uj, uj], axis=0)

    rr = jax.lax.broadcasted_iota(jnp.int32, (_ROWS, _ROWS), 0)
    cc = jax.lax.broadcasted_iota(jnp.int32, (_ROWS, _ROWS), 1)
    ll = (rr > cc).astype(jnp.float32)

    x_hi, x_lo = _split(x)
    xx = jnp.concatenate([x_hi, x_lo], axis=1)
    m1 = jax.lax.dot(xx, ujuj, preferred_element_type=jnp.float32)
    c = m1[:, :_COLS]
    t = m1[:, _COLS:]

    t_hi = t.astype(jnp.bfloat16).astype(jnp.float32)
    prefix = jax.lax.dot(ll, t_hi, preferred_element_type=jnp.float32)

    o_ref[:] = c + prefix


def kernel(input_0):
    x = input_0.reshape(_ROWS, _COLS)
    out = pl.pallas_call(
        _cumsum_body,
        out_shape=jax.ShapeDtypeStruct((_ROWS, _COLS), jnp.float32),
        in_specs=[pl.BlockSpec((_ROWS, _COLS), lambda: (0, 0))],
        out_specs=pl.BlockSpec((_ROWS, _COLS), lambda: (0, 0)),
        input_output_aliases={0: 0},
    )(x)
    return out.reshape(32768)


# speedup vs baseline: 1.6199x; 1.6199x over previous
import jax
import jax.numpy as jnp
from jax.experimental import pallas as pl
from jax.experimental.pallas import tpu as pltpu

_ROWS = 256
_COLS = 128
_G = 2
_BR = _ROWS // _G


def _split(a):
    hi = a.astype(jnp.bfloat16).astype(jnp.float32)
    return hi, a - hi


def _body(x_ref, o_ref, carry_ref):
    k = pl.program_id(0)
    x = x_ref[:]

    ii = jax.lax.broadcasted_iota(jnp.int32, (_COLS, 2 * _COLS), 0)
    jj = jax.lax.broadcasted_iota(jnp.int32, (_COLS, 2 * _COLS), 1)
    uj = ((ii <= jj) | (jj >= _COLS)).astype(jnp.float32)
    ujuj = jnp.concatenate([uj, uj], axis=0)

    rr = jax.lax.broadcasted_iota(jnp.int32, (_BR, _BR), 0)
    cc = jax.lax.broadcasted_iota(jnp.int32, (_BR, _BR), 1)
    ll = (rr > cc).astype(jnp.float32)

    x_hi, x_lo = _split(x)
    xx = jnp.concatenate([x_hi, x_lo], axis=1)
    m1 = jax.lax.dot(xx, ujuj, preferred_element_type=jnp.float32)
    c = m1[:, :_COLS]
    t = m1[:, _COLS:]

    t_hi = t.astype(jnp.bfloat16).astype(jnp.float32)
    p = jax.lax.dot(ll, t_hi, preferred_element_type=jnp.float32)

    @pl.when(k == 0)
    def _():
        carry_ref[...] = jnp.zeros((8, _COLS), jnp.float32)

    carry = carry_ref[0:1, :]
    o_ref[:] = c + p + carry
    carry_ref[0:1, :] = carry + p[_BR - 1:_BR, :] + t[_BR - 1:_BR, :]


def kernel(input_0):
    x = input_0.reshape(_ROWS, _COLS)
    out = pl.pallas_call(
        _body,
        grid=(_G,),
        out_shape=jax.ShapeDtypeStruct((_ROWS, _COLS), jnp.float32),
        in_specs=[pl.BlockSpec((_BR, _COLS), lambda k: (k, 0))],
        out_specs=pl.BlockSpec((_BR, _COLS), lambda k: (k, 0)),
        scratch_shapes=[pltpu.VMEM((8, _COLS), jnp.float32)],
    )(x)
    return out.reshape(32768)


# speedup vs baseline: 1.7803x; 1.0990x over previous
import jax
import jax.numpy as jnp
from jax.experimental import pallas as pl

_ROWS = 256
_COLS = 128


def _split(a):
    hi = a.astype(jnp.bfloat16).astype(jnp.float32)
    return hi, a - hi


def _cumsum_body(x_ref, o_ref):
    x = x_ref[:]

    ii = jax.lax.broadcasted_iota(jnp.int32, (_COLS, 2 * _COLS), 0)
    jj = jax.lax.broadcasted_iota(jnp.int32, (_COLS, 2 * _COLS), 1)
    uj = ((ii <= jj) | (jj >= _COLS)).astype(jnp.bfloat16)
    ujuj = jnp.concatenate([uj, uj], axis=0)

    rr = jax.lax.broadcasted_iota(jnp.int32, (_ROWS, _ROWS), 0)
    cc = jax.lax.broadcasted_iota(jnp.int32, (_ROWS, _ROWS), 1)
    ll = (rr > cc).astype(jnp.bfloat16)

    x_hi = x.astype(jnp.bfloat16)
    x_lo = (x - x_hi.astype(jnp.float32)).astype(jnp.bfloat16)
    xx = jnp.concatenate([x_hi, x_lo], axis=1)
    m1 = jax.lax.dot(xx, ujuj, preferred_element_type=jnp.float32)
    c = m1[:, :_COLS]
    t = m1[:, _COLS:]

    t_hi = t.astype(jnp.bfloat16)
    prefix = jax.lax.dot(ll, t_hi, preferred_element_type=jnp.float32)

    o_ref[:] = c + prefix


def kernel(input_0):
    x = input_0.reshape(_ROWS, _COLS)
    out = pl.pallas_call(
        _cumsum_body,
        out_shape=jax.ShapeDtypeStruct((_ROWS, _COLS), jnp.float32),
        in_specs=[pl.BlockSpec((_ROWS, _COLS), lambda: (0, 0))],
        out_specs=pl.BlockSpec((_ROWS, _COLS), lambda: (0, 0)),
    )(x)
    return out.reshape(32768)
